# baseline (speedup 1.0000x reference)
import functools

import jax
import jax.numpy as jnp
from jax import lax
from jax.experimental import pallas as pl
from jax.experimental.pallas import tpu as pltpu
from jax.experimental.pallas import tpu_sc as plsc

_N = 50000
_E = 800000
_IN = 64
_H = 256
_OUT = 128
_B = 512
_EPS = 1e-5

_NP = 51200
_NC = 2
_NS = 16
_EPAD = 819200
_EPT = _EPAD // _NS
_CE = 1600
_CNTP = 640
_CNTC = 768


def _sc_mesh():
    return plsc.VectorSubcoreMesh(core_axis_name="c", subcore_axis_name="s",
                                  num_cores=_NC, num_subcores=_NS)


def _deg_body(dst3, degp, dbuf, onesv, zbuf, acc, sem):
    c = lax.axis_index("c")
    s = lax.axis_index("s")

    def _zero16(i, _):
        zbuf[pl.ds(i * 16, 16)] = jnp.zeros((16,), jnp.float32)
        return 0
    lax.fori_loop(0, 3200 // 16, _zero16, 0)
    for k in range(8):
        onesv[pl.ds(k * 16, 16)] = jnp.ones((16,), jnp.float32)
    pltpu.sync_copy(zbuf, acc.at[pl.ds(s * 3200, 3200)])
    plsc.subcore_barrier()

    r0 = c * 3200 + s * 200
    for ch in range(5):
        pltpu.sync_copy(dst3.at[pl.ds(r0 + ch * 40, 40)], dbuf)

        def _add_row(j, _):
            pltpu.sync_copy(onesv, acc.at[dbuf.at[j]], add=True)
            return 0
        lax.fori_loop(0, 40, _add_row, 0)
    plsc.subcore_barrier()
    pltpu.sync_copy(acc.at[pl.ds(s * 3200, 3200)], zbuf)
    pltpu.sync_copy(zbuf, degp.at[c, pl.ds(s * 3200, 3200)])


@functools.cache
def _deg_kernel():
    return pl.kernel(
        _deg_body,
        out_type=jax.ShapeDtypeStruct((_NC, _NP), jnp.float32),
        mesh=_sc_mesh(),
        scratch_types=[
            pltpu.VMEM((40, 128), jnp.int32),
            pltpu.VMEM((128,), jnp.float32),
            pltpu.VMEM((3200,), jnp.float32),
            pltpu.VMEM_SHARED((_NP,), jnp.float32),
            pltpu.SemaphoreType.DMA,
        ],
    )


def _msg_body(NB, src_h, dst_h, u_h, y_h, srcb, dstb, qsrc, qdst, rowb, acc, sem):
    c = lax.axis_index("c")
    s = lax.axis_index("s")
    bpc = (_NP // NB) // _NC
    stride = NB // _NS
    iota16 = lax.iota(jnp.int32, 16)

    for blk in range(bpc):
        row_lo = (c * bpc + blk) * NB

        def _init(q, _):
            pltpu.sync_copy(u_h.at[pl.ds(row_lo + s * stride + q * 80, 80)],
                            rowb)
            pltpu.sync_copy(rowb, acc.at[pl.ds(s * stride + q * 80, 80)])
            return 0
        lax.fori_loop(0, stride // 80, _init, 0)
        plsc.subcore_barrier()

        def _chunk(ch, _):
            ebase = s * _EPT + ch * _CE
            pltpu.sync_copy(src_h.at[pl.ds(ebase, _CE)], srcb)
            pltpu.sync_copy(dst_h.at[pl.ds(ebase, _CE)], dstb)

            def _group(fq, _):
                for k in range(5):
                    d = dstb[pl.ds(fq * 80 + k * 16, 16)]
                    sv = srcb[pl.ds(fq * 80 + k * 16, 16)]
                    dl = d - row_lo
                    inb = (dl >= 0) & (dl < NB)
                    qdst[pl.ds(k * 16, 16)] = jnp.where(inb, dl, NB + iota16)
                    qsrc[pl.ds(k * 16, 16)] = sv
                pltpu.async_copy(u_h.at[qsrc], rowb, sem).wait()
                pltpu.sync_copy(rowb, acc.at[qdst], add=True)
                return 0

            lax.fori_loop(0, _CE // 80, _group, 0)
            return 0

        lax.fori_loop(0, _EPT // _CE, _chunk, 0)

        plsc.subcore_barrier()

        def _wout(q, _):
            pltpu.sync_copy(acc.at[pl.ds(s * stride + q * 80, 80)], rowb)
            pltpu.sync_copy(rowb,
                            y_h.at[pl.ds(row_lo + s * stride + q * 80, 80)])
            return 0
        lax.fori_loop(0, stride // 80, _wout, 0)
        plsc.subcore_barrier()


@functools.cache
def _msg_kernel(D, NB):
    return pl.kernel(
        functools.partial(_msg_body, NB),
        out_type=jax.ShapeDtypeStruct((_NP, D), jnp.float32),
        mesh=_sc_mesh(),
        scratch_types=[
            pltpu.VMEM((_CE,), jnp.int32),
            pltpu.VMEM((_CE,), jnp.int32),
            pltpu.VMEM((80,), jnp.int32),
            pltpu.VMEM((80,), jnp.int32),
            pltpu.VMEM((80, D), jnp.float32),
            pltpu.VMEM_SHARED((NB + 16, D), jnp.float32),
            pltpu.SemaphoreType.DMA,
        ],
    )


def _pool_body(x4_h, batch_h, z2_h, z1_h, pool_h, cnt_h, xbuf, bidx, onesb, cbuf, pacc, cacc, sem):
    c = lax.axis_index("c")
    s = lax.axis_index("s")

    for k in range(5):
        onesb[pl.ds(k * 16, 16)] = jnp.ones((16,), jnp.float32)

    pltpu.sync_copy(z2_h.at[pl.ds(s * 40, 40)], xbuf.at[pl.ds(0, 40)])
    pltpu.sync_copy(xbuf.at[pl.ds(0, 40)], pacc.at[pl.ds(s * 40, 40)])
    pltpu.sync_copy(z1_h.at[pl.ds(s * 48, 48)], cbuf)
    pltpu.sync_copy(cbuf, cacc.at[pl.ds(s * 48, 48)])
    plsc.subcore_barrier()

    base0 = (c * _NS + s) * 1600
    for ch in range(20):
        base = base0 + ch * 80
        pltpu.sync_copy(x4_h.at[pl.ds(base, 80)], xbuf)
        pltpu.sync_copy(batch_h.at[pl.ds(base, 80)], bidx)
        pltpu.sync_copy(xbuf, pacc.at[bidx], add=True)
        pltpu.sync_copy(onesb, cacc.at[bidx], add=True)
    plsc.subcore_barrier()
    pltpu.sync_copy(pacc.at[pl.ds(s * 40, 40)], xbuf.at[pl.ds(0, 40)])
    pltpu.sync_copy(xbuf.at[pl.ds(0, 40)],
                    pool_h.at[pl.ds(c * _CNTP + s * 40, 40)])
    pltpu.sync_copy(cacc.at[pl.ds(s * 48, 48)], cbuf)
    pltpu.sync_copy(cbuf, cnt_h.at[pl.ds(c * _CNTC + s * 48, 48)])


@functools.cache
def _pool_kernel():
    return pl.kernel(
        _pool_body,
        out_type=(jax.ShapeDtypeStruct((_NC * _CNTP, _OUT), jnp.float32),
                  jax.ShapeDtypeStruct((_NC * _CNTC,), jnp.float32)),
        mesh=_sc_mesh(),
        scratch_types=[
            pltpu.VMEM((80, _OUT), jnp.float32),
            pltpu.VMEM((80,), jnp.int32),
            pltpu.VMEM((80,), jnp.float32),
            pltpu.VMEM((48,), jnp.float32),
            pltpu.VMEM_SHARED((_CNTP, _OUT), jnp.float32),
            pltpu.VMEM_SHARED((_CNTC,), jnp.float32),
            pltpu.SemaphoreType.DMA,
        ],
    )


def _dinv_body(dp_ref, o_ref):
    d = dp_ref[0] + dp_ref[1] + 1.0
    r = lax.rsqrt(d)
    row = lax.broadcasted_iota(jnp.int32, (_NP // 128, 128), 0)
    col = lax.broadcasted_iota(jnp.int32, (_NP // 128, 128), 1)
    o_ref[...] = jnp.where(row * 128 + col < _N, r, 0.0)


@functools.cache
def _dinv_kernel():
    return pl.pallas_call(
        _dinv_body,
        out_shape=jax.ShapeDtypeStruct((_NP // 128, 128), jnp.float32),
    )


def _rowscale_body(x_ref, dv_ref, o_ref):
    xb = x_ref[...] * dv_ref[...]
    o_ref[...] = jnp.concatenate(
        [xb, jnp.zeros((xb.shape[0], _OUT - _IN), jnp.float32)], axis=1)


@functools.cache
def _rowscale_kernel(R):
    return pl.pallas_call(
        _rowscale_body,
        grid=(_NP // R,),
        in_specs=[pl.BlockSpec((R, _IN), lambda i: (i, 0)),
                  pl.BlockSpec((R, 1), lambda i: (i, 0))],
        out_specs=pl.BlockSpec((R, _OUT), lambda i: (i, 0)),
        out_shape=jax.ShapeDtypeStruct((_NP, _OUT), jnp.float32),
    )


def _mm_stats_body(g_ref, dv_ref, w_ref, z_ref, st_ref):
    i = pl.program_id(0)
    v = g_ref[...] * dv_ref[...]
    z = jnp.dot(v, w_ref[...], preferred_element_type=jnp.float32)
    z_ref[...] = z

    @pl.when(i == 0)
    def _():
        st_ref[...] = jnp.zeros_like(st_ref)

    st_ref[0:1, :] = st_ref[0:1, :] + jnp.sum(z, axis=0, keepdims=True)
    st_ref[1:2, :] = st_ref[1:2, :] + jnp.sum(z * z, axis=0, keepdims=True)


@functools.cache
def _mm_stats_kernel(Din, Dout, R):
    return pl.pallas_call(
        _mm_stats_body,
        grid=(_NP // R,),
        in_specs=[pl.BlockSpec((R, Din), lambda i: (i, 0)),
                  pl.BlockSpec((R, 1), lambda i: (i, 0)),
                  pl.BlockSpec((Din, Dout), lambda i: (0, 0))],
        out_specs=[pl.BlockSpec((R, Dout), lambda i: (i, 0)),
                   pl.BlockSpec((8, Dout), lambda i: (0, 0))],
        out_shape=[jax.ShapeDtypeStruct((_NP, Dout), jnp.float32),
                   jax.ShapeDtypeStruct((8, Dout), jnp.float32)],
    )


def _mm2_stats_body(ga_ref, gb_ref, dv_ref, wa_ref, wb_ref, z_ref, st_ref):
    i = pl.program_id(0)
    dv = dv_ref[...]
    z = jnp.dot(ga_ref[...] * dv, wa_ref[...],
                preferred_element_type=jnp.float32)
    z = z + jnp.dot(gb_ref[...] * dv, wb_ref[...],
                    preferred_element_type=jnp.float32)
    z_ref[...] = z

    @pl.when(i == 0)
    def _():
        st_ref[...] = jnp.zeros_like(st_ref)

    st_ref[0:1, :] = st_ref[0:1, :] + jnp.sum(z, axis=0, keepdims=True)
    st_ref[1:2, :] = st_ref[1:2, :] + jnp.sum(z * z, axis=0, keepdims=True)


@functools.cache
def _mm2_stats_kernel(R):
    half = _H // 2
    return pl.pallas_call(
        _mm2_stats_body,
        grid=(_NP // R,),
        in_specs=[pl.BlockSpec((R, half), lambda i: (i, 0)),
                  pl.BlockSpec((R, half), lambda i: (i, 0)),
                  pl.BlockSpec((R, 1), lambda i: (i, 0)),
                  pl.BlockSpec((half, _H), lambda i: (0, 0)),
                  pl.BlockSpec((half, _H), lambda i: (0, 0))],
        out_specs=[pl.BlockSpec((R, _H), lambda i: (i, 0)),
                   pl.BlockSpec((8, _H), lambda i: (0, 0))],
        out_shape=[jax.ShapeDtypeStruct((_NP, _H), jnp.float32),
                   jax.ShapeDtypeStruct((8, _H), jnp.float32)],
    )


def _bn_coeffs(st_ref, gg_ref, be_ref):
    m = st_ref[0:1, :] * (1.0 / _N)
    v = st_ref[1:2, :] * (1.0 / _N) - m * m
    a = gg_ref[...] * lax.rsqrt(v + _EPS)
    return a, be_ref[...] - m * a


def _bnrelu_scale_body(z_ref, st_ref, gg_ref, be_ref, dv_ref, oa_ref, ob_ref):
    a, b = _bn_coeffs(st_ref, gg_ref, be_ref)
    v = jnp.maximum(z_ref[...] * a + b, 0.0) * dv_ref[...]
    half = _H // 2
    oa_ref[...] = v[:, :half]
    ob_ref[...] = v[:, half:]


@functools.cache
def _bnrelu_scale_kernel(R):
    half = _H // 2
    return pl.pallas_call(
        _bnrelu_scale_body,
        grid=(_NP // R,),
        in_specs=[pl.BlockSpec((R, _H), lambda i: (i, 0)),
                  pl.BlockSpec((8, _H), lambda i: (0, 0)),
                  pl.BlockSpec((1, _H), lambda i: (0, 0)),
                  pl.BlockSpec((1, _H), lambda i: (0, 0)),
                  pl.BlockSpec((R, 1), lambda i: (i, 0))],
        out_specs=[pl.BlockSpec((R, half), lambda i: (i, 0)),
                   pl.BlockSpec((R, half), lambda i: (i, 0))],
        out_shape=[jax.ShapeDtypeStruct((_NP, half), jnp.float32),
                   jax.ShapeDtypeStruct((_NP, half), jnp.float32)],
    )


def _bnrelu_mm_body(z_ref, st_ref, gg_ref, be_ref, dv_ref, w_ref, o_ref):
    a, b = _bn_coeffs(st_ref, gg_ref, be_ref)
    xn = jnp.maximum(z_ref[...] * a + b, 0.0)
    o_ref[...] = jnp.dot(xn, w_ref[...],
                         preferred_element_type=jnp.float32) * dv_ref[...]


@functools.cache
def _bnrelu_mm_kernel(Din, Dout, R):
    return pl.pallas_call(
        _bnrelu_mm_body,
        grid=(_NP // R,),
        in_specs=[pl.BlockSpec((R, Din), lambda i: (i, 0)),
                  pl.BlockSpec((8, Din), lambda i: (0, 0)),
                  pl.BlockSpec((1, Din), lambda i: (0, 0)),
                  pl.BlockSpec((1, Din), lambda i: (0, 0)),
                  pl.BlockSpec((R, 1), lambda i: (i, 0)),
                  pl.BlockSpec((Din, Dout), lambda i: (0, 0))],
        out_specs=pl.BlockSpec((R, Dout), lambda i: (i, 0)),
        out_shape=jax.ShapeDtypeStruct((_NP, Dout), jnp.float32),
    )


def _stats_scaled_body(y_ref, dv_ref, st_ref):
    i = pl.program_id(0)
    z = y_ref[...] * dv_ref[...]

    @pl.when(i == 0)
    def _():
        st_ref[...] = jnp.zeros_like(st_ref)

    st_ref[0:1, :] = st_ref[0:1, :] + jnp.sum(z, axis=0, keepdims=True)
    st_ref[1:2, :] = st_ref[1:2, :] + jnp.sum(z * z, axis=0, keepdims=True)


@functools.cache
def _stats_scaled_kernel(D, R):
    return pl.pallas_call(
        _stats_scaled_body,
        grid=(_NP // R,),
        in_specs=[pl.BlockSpec((R, D), lambda i: (i, 0)),
                  pl.BlockSpec((R, 1), lambda i: (i, 0))],
        out_specs=pl.BlockSpec((8, D), lambda i: (0, 0)),
        out_shape=jax.ShapeDtypeStruct((8, D), jnp.float32),
    )


def _bnrelu_scaled_body(y_ref, st_ref, gg_ref, be_ref, dv_ref, o_ref):
    a, b = _bn_coeffs(st_ref, gg_ref, be_ref)
    o_ref[...] = jnp.maximum((y_ref[...] * dv_ref[...]) * a + b, 0.0)


@functools.cache
def _bnrelu_scaled_kernel(D, R):
    return pl.pallas_call(
        _bnrelu_scaled_body,
        grid=(_NP // R,),
        in_specs=[pl.BlockSpec((R, D), lambda i: (i, 0)),
                  pl.BlockSpec((8, D), lambda i: (0, 0)),
                  pl.BlockSpec((1, D), lambda i: (0, 0)),
                  pl.BlockSpec((1, D), lambda i: (0, 0)),
                  pl.BlockSpec((R, 1), lambda i: (i, 0))],
        out_specs=pl.BlockSpec((R, D), lambda i: (i, 0)),
        out_shape=jax.ShapeDtypeStruct((_NP, D), jnp.float32),
    )


def _final_body(pp_ref, cc_ref, o_ref):
    p = pp_ref[0, :_B, :] + pp_ref[1, :_B, :]
    cnt = cc_ref[0, :_B, :] + cc_ref[1, :_B, :]
    p = p / jnp.maximum(cnt, 1.0)
    nrm = jnp.sqrt(jnp.sum(p * p, axis=1, keepdims=True))
    o_ref[...] = p / jnp.maximum(nrm, 1e-12)


@functools.cache
def _final_kernel():
    return pl.pallas_call(
        _final_body,
        out_shape=jax.ShapeDtypeStruct((_B, _OUT), jnp.float32),
    )


def kernel(x, edge_index, batch, W1, b1, g1, be1, W2, b2, g2, be2, W3, b3, g3, be3):
    f32 = jnp.float32
    src = edge_index[0]
    dst = edge_index[1]

    xp = jnp.pad(x, ((0, _NP - _N), (0, 0)))
    padfill = (jnp.arange(_EPAD - _E) % 1024).astype(jnp.int32)
    src_pad = jnp.concatenate([src, padfill])
    dst_pad = jnp.concatenate([dst, _N + padfill])
    batch_pad = jnp.concatenate(
        [batch, (_B + (jnp.arange(_NP - _N) % 16)).astype(jnp.int32)])

    degp = _deg_kernel()(dst_pad.reshape(_EPAD // 128, 128))
    dinv = _dinv_kernel()(degp.reshape(_NC, _NP // 128, 128)).reshape(_NP, 1)

    msg = _msg_kernel(128, 12800)

    V1 = _rowscale_kernel(6400)(xp, dinv)
    G1 = msg(src_pad, dst_pad, V1)
    W1p = jnp.pad(W1, ((0, _OUT - _IN), (0, 0)))
    Z1, st1 = _mm_stats_kernel(_OUT, _H, 3200)(G1, dinv, W1p)

    V2a, V2b = _bnrelu_scale_kernel(3200)(Z1, st1, g1.reshape(1, _H),
                                          be1.reshape(1, _H), dinv)
    G2a = msg(src_pad, dst_pad, V2a)
    G2b = msg(src_pad, dst_pad, V2b)
    Z2, st2 = _mm2_stats_kernel(3200)(G2a, G2b, dinv, W2[:_H // 2],
                                      W2[_H // 2:])

    U3 = _bnrelu_mm_kernel(_H, _OUT, 3200)(Z2, st2, g2.reshape(1, _H),
                                           be2.reshape(1, _H), dinv, W3)
    Y3 = msg(src_pad, dst_pad, U3)
    st3 = _stats_scaled_kernel(_OUT, 6400)(Y3, dinv)
    X4 = _bnrelu_scaled_kernel(_OUT, 6400)(Y3, st3, g3.reshape(1, _OUT),
                                           be3.reshape(1, _OUT), dinv)

    poolp, cntp = _pool_kernel()(
        X4, batch_pad,
        jnp.zeros((_CNTP, _OUT), f32), jnp.zeros((_CNTC,), f32))
    return _final_kernel()(poolp.reshape(_NC, _CNTP, _OUT),
                           cntp.reshape(_NC, _CNTC, 1))


# speedup vs baseline: 1.6669x; 1.6669x over previous
import functools

import jax
import jax.numpy as jnp
from jax import lax
from jax.experimental import pallas as pl
from jax.experimental.pallas import tpu as pltpu
from jax.experimental.pallas import tpu_sc as plsc

_N = 50000
_E = 800000
_IN = 64
_H = 256
_OUT = 128
_B = 512
_EPS = 1e-5

_NP = 51200
_NC = 2
_NS = 16
_EPAD = 819200
_EPT = _EPAD // _NS
_CE = 1600
_QDUMP = _CE + 128
_QCAP = _QDUMP + 16
_CNTP = 640
_CNTC = 768


def _sc_mesh():
    return plsc.VectorSubcoreMesh(core_axis_name="c", subcore_axis_name="s",
                                  num_cores=_NC, num_subcores=_NS)


def _deg_body(dst3, degp, dbuf, onesv, zbuf, acc, sem):
    c = lax.axis_index("c")
    s = lax.axis_index("s")

    def _zero16(i, _):
        zbuf[pl.ds(i * 16, 16)] = jnp.zeros((16,), jnp.float32)
        return 0
    lax.fori_loop(0, 3200 // 16, _zero16, 0)
    for k in range(8):
        onesv[pl.ds(k * 16, 16)] = jnp.ones((16,), jnp.float32)
    pltpu.sync_copy(zbuf, acc.at[pl.ds(s * 3200, 3200)])
    plsc.subcore_barrier()

    r0 = c * 3200 + s * 200
    for ch in range(5):
        pltpu.sync_copy(dst3.at[pl.ds(r0 + ch * 40, 40)], dbuf)

        def _add_row(j, _):
            pltpu.sync_copy(onesv, acc.at[dbuf.at[j]], add=True)
            return 0
        lax.fori_loop(0, 40, _add_row, 0)
    plsc.subcore_barrier()
    pltpu.sync_copy(acc.at[pl.ds(s * 3200, 3200)], zbuf)
    pltpu.sync_copy(zbuf, degp.at[c, pl.ds(s * 3200, 3200)])


@functools.cache
def _deg_kernel():
    return pl.kernel(
        _deg_body,
        out_type=jax.ShapeDtypeStruct((_NC, _NP), jnp.float32),
        mesh=_sc_mesh(),
        scratch_types=[
            pltpu.VMEM((40, 128), jnp.int32),
            pltpu.VMEM((128,), jnp.float32),
            pltpu.VMEM((3200,), jnp.float32),
            pltpu.VMEM_SHARED((_NP,), jnp.float32),
            pltpu.SemaphoreType.DMA,
        ],
    )


def _msg_body(NB, src_h, dst_h, u_h, y_h, srcb, dstb, qsb, qdb, qsb2, qdb2, rowb, rowb2, acc, sem, sem2):
    c = lax.axis_index("c")
    s = lax.axis_index("s")
    bpc = (_NP // NB) // _NC
    stride = NB // _NS
    iota16 = lax.iota(jnp.int32, 16)

    for blk in range(bpc):
        row_lo = (c * bpc + blk) * NB

        def _init(q, _):
            pltpu.sync_copy(u_h.at[pl.ds(row_lo + s * stride + q * 80, 80)],
                            rowb.at[pl.ds(0, 80)])
            pltpu.sync_copy(rowb.at[pl.ds(0, 80)],
                            acc.at[pl.ds(s * stride + q * 80, 80)])
            return 0
        lax.fori_loop(0, stride // 80, _init, 0)
        plsc.subcore_barrier()

        def _chunk(ch, _):
            ebase = s * _EPT + ch * _CE
            pltpu.sync_copy(src_h.at[pl.ds(ebase, _CE)], srcb)
            pltpu.sync_copy(dst_h.at[pl.ds(ebase, _CE)], dstb)

            ngrp = _CE // 80
            qsbs = (qsb, qsb2)
            qdbs = (qdb, qdb2)
            rows = (rowb, rowb2)
            sems = (sem, sem2)

            def _build(g, qs, qd):
                for k in range(5):
                    d = dstb[pl.ds(g * 80 + k * 16, 16)]
                    sv = srcb[pl.ds(g * 80 + k * 16, 16)]
                    dl = d - row_lo
                    inb = (dl >= 0) & (dl < NB)
                    qd[pl.ds(k * 16, 16)] = jnp.where(inb, dl, NB + iota16)
                    qs[pl.ds(k * 16, 16)] = sv

            _build(0, qsbs[0], qdbs[0])
            prev = pltpu.async_copy(u_h.at[qsbs[0]], rows[0], sems[0])
            for g in range(1, ngrp):
                p = g & 1
                _build(g, qsbs[p], qdbs[p])
                cur = pltpu.async_copy(u_h.at[qsbs[p]], rows[p], sems[p])
                prev.wait()
                pltpu.sync_copy(rows[1 - p], acc.at[qdbs[1 - p]], add=True)
                prev = cur
            prev.wait()
            pltpu.sync_copy(rows[(ngrp - 1) & 1],
                            acc.at[qdbs[(ngrp - 1) & 1]], add=True)
            return 0

        lax.fori_loop(0, _EPT // _CE, _chunk, 0)

        plsc.subcore_barrier()

        def _wout(q, _):
            pltpu.sync_copy(acc.at[pl.ds(s * stride + q * 80, 80)],
                            rowb.at[pl.ds(0, 80)])
            pltpu.sync_copy(rowb.at[pl.ds(0, 80)],
                            y_h.at[pl.ds(row_lo + s * stride + q * 80, 80)])
            return 0
        lax.fori_loop(0, stride // 80, _wout, 0)
        plsc.subcore_barrier()


@functools.cache
def _msg_kernel(D, NB):
    return pl.kernel(
        functools.partial(_msg_body, NB),
        out_type=jax.ShapeDtypeStruct((_NP, D), jnp.float32),
        mesh=_sc_mesh(),
        scratch_types=[
            pltpu.VMEM((_CE,), jnp.int32),
            pltpu.VMEM((_CE,), jnp.int32),
            pltpu.VMEM((80,), jnp.int32),
            pltpu.VMEM((80,), jnp.int32),
            pltpu.VMEM((80,), jnp.int32),
            pltpu.VMEM((80,), jnp.int32),
            pltpu.VMEM((80, D), jnp.float32),
            pltpu.VMEM((80, D), jnp.float32),
            pltpu.VMEM_SHARED((NB + 16, D), jnp.float32),
            pltpu.SemaphoreType.DMA,
            pltpu.SemaphoreType.DMA,
        ],
    )


def _pool_body(x4_h, batch_h, z2_h, z1_h, pool_h, cnt_h, xbuf, bidx, onesb, cbuf, pacc, cacc, sem):
    c = lax.axis_index("c")
    s = lax.axis_index("s")

    for k in range(5):
        onesb[pl.ds(k * 16, 16)] = jnp.ones((16,), jnp.float32)

    pltpu.sync_copy(z2_h.at[pl.ds(s * 40, 40)], xbuf.at[pl.ds(0, 40)])
    pltpu.sync_copy(xbuf.at[pl.ds(0, 40)], pacc.at[pl.ds(s * 40, 40)])
    pltpu.sync_copy(z1_h.at[pl.ds(s * 48, 48)], cbuf)
    pltpu.sync_copy(cbuf, cacc.at[pl.ds(s * 48, 48)])
    plsc.subcore_barrier()

    base0 = (c * _NS + s) * 1600
    for ch in range(20):
        base = base0 + ch * 80
        pltpu.sync_copy(x4_h.at[pl.ds(base, 80)], xbuf)
        pltpu.sync_copy(batch_h.at[pl.ds(base, 80)], bidx)
        pltpu.sync_copy(xbuf, pacc.at[bidx], add=True)
        pltpu.sync_copy(onesb, cacc.at[bidx], add=True)
    plsc.subcore_barrier()
    pltpu.sync_copy(pacc.at[pl.ds(s * 40, 40)], xbuf.at[pl.ds(0, 40)])
    pltpu.sync_copy(xbuf.at[pl.ds(0, 40)],
                    pool_h.at[pl.ds(c * _CNTP + s * 40, 40)])
    pltpu.sync_copy(cacc.at[pl.ds(s * 48, 48)], cbuf)
    pltpu.sync_copy(cbuf, cnt_h.at[pl.ds(c * _CNTC + s * 48, 48)])


@functools.cache
def _pool_kernel():
    return pl.kernel(
        _pool_body,
        out_type=(jax.ShapeDtypeStruct((_NC * _CNTP, _OUT), jnp.float32),
                  jax.ShapeDtypeStruct((_NC * _CNTC,), jnp.float32)),
        mesh=_sc_mesh(),
        scratch_types=[
            pltpu.VMEM((80, _OUT), jnp.float32),
            pltpu.VMEM((80,), jnp.int32),
            pltpu.VMEM((80,), jnp.float32),
            pltpu.VMEM((48,), jnp.float32),
            pltpu.VMEM_SHARED((_CNTP, _OUT), jnp.float32),
            pltpu.VMEM_SHARED((_CNTC,), jnp.float32),
            pltpu.SemaphoreType.DMA,
        ],
    )


def _dinv_body(dp_ref, o_ref):
    d = dp_ref[0] + dp_ref[1] + 1.0
    r = lax.rsqrt(d)
    row = lax.broadcasted_iota(jnp.int32, (_NP // 128, 128), 0)
    col = lax.broadcasted_iota(jnp.int32, (_NP // 128, 128), 1)
    o_ref[...] = jnp.where(row * 128 + col < _N, r, 0.0)


@functools.cache
def _dinv_kernel():
    return pl.pallas_call(
        _dinv_body,
        out_shape=jax.ShapeDtypeStruct((_NP // 128, 128), jnp.float32),
    )


def _rowscale_body(x_ref, dv_ref, o_ref):
    xb = x_ref[...] * dv_ref[...]
    o_ref[...] = jnp.concatenate(
        [xb, jnp.zeros((xb.shape[0], _OUT - _IN), jnp.float32)], axis=1)


@functools.cache
def _rowscale_kernel(R):
    return pl.pallas_call(
        _rowscale_body,
        grid=(_NP // R,),
        in_specs=[pl.BlockSpec((R, _IN), lambda i: (i, 0)),
                  pl.BlockSpec((R, 1), lambda i: (i, 0))],
        out_specs=pl.BlockSpec((R, _OUT), lambda i: (i, 0)),
        out_shape=jax.ShapeDtypeStruct((_NP, _OUT), jnp.float32),
    )


def _mm_stats_body(g_ref, dv_ref, w_ref, z_ref, st_ref):
    i = pl.program_id(0)
    v = g_ref[...] * dv_ref[...]
    z = jnp.dot(v, w_ref[...], preferred_element_type=jnp.float32)
    z_ref[...] = z

    @pl.when(i == 0)
    def _():
        st_ref[...] = jnp.zeros_like(st_ref)

    st_ref[0:1, :] = st_ref[0:1, :] + jnp.sum(z, axis=0, keepdims=True)
    st_ref[1:2, :] = st_ref[1:2, :] + jnp.sum(z * z, axis=0, keepdims=True)


@functools.cache
def _mm_stats_kernel(Din, Dout, R):
    return pl.pallas_call(
        _mm_stats_body,
        grid=(_NP // R,),
        in_specs=[pl.BlockSpec((R, Din), lambda i: (i, 0)),
                  pl.BlockSpec((R, 1), lambda i: (i, 0)),
                  pl.BlockSpec((Din, Dout), lambda i: (0, 0))],
        out_specs=[pl.BlockSpec((R, Dout), lambda i: (i, 0)),
                   pl.BlockSpec((8, Dout), lambda i: (0, 0))],
        out_shape=[jax.ShapeDtypeStruct((_NP, Dout), jnp.float32),
                   jax.ShapeDtypeStruct((8, Dout), jnp.float32)],
    )


def _mm2_stats_body(ga_ref, gb_ref, dv_ref, wa_ref, wb_ref, z_ref, st_ref):
    i = pl.program_id(0)
    dv = dv_ref[...]
    z = jnp.dot(ga_ref[...] * dv, wa_ref[...],
                preferred_element_type=jnp.float32)
    z = z + jnp.dot(gb_ref[...] * dv, wb_ref[...],
                    preferred_element_type=jnp.float32)
    z_ref[...] = z

    @pl.when(i == 0)
    def _():
        st_ref[...] = jnp.zeros_like(st_ref)

    st_ref[0:1, :] = st_ref[0:1, :] + jnp.sum(z, axis=0, keepdims=True)
    st_ref[1:2, :] = st_ref[1:2, :] + jnp.sum(z * z, axis=0, keepdims=True)


@functools.cache
def _mm2_stats_kernel(R):
    half = _H // 2
    return pl.pallas_call(
        _mm2_stats_body,
        grid=(_NP // R,),
        in_specs=[pl.BlockSpec((R, half), lambda i: (i, 0)),
                  pl.BlockSpec((R, half), lambda i: (i, 0)),
                  pl.BlockSpec((R, 1), lambda i: (i, 0)),
                  pl.BlockSpec((half, _H), lambda i: (0, 0)),
                  pl.BlockSpec((half, _H), lambda i: (0, 0))],
        out_specs=[pl.BlockSpec((R, _H), lambda i: (i, 0)),
                   pl.BlockSpec((8, _H), lambda i: (0, 0))],
        out_shape=[jax.ShapeDtypeStruct((_NP, _H), jnp.float32),
                   jax.ShapeDtypeStruct((8, _H), jnp.float32)],
    )


def _bn_coeffs(st_ref, gg_ref, be_ref):
    m = st_ref[0:1, :] * (1.0 / _N)
    v = st_ref[1:2, :] * (1.0 / _N) - m * m
    a = gg_ref[...] * lax.rsqrt(v + _EPS)
    return a, be_ref[...] - m * a


def _bnrelu_scale_body(z_ref, st_ref, gg_ref, be_ref, dv_ref, oa_ref, ob_ref):
    a, b = _bn_coeffs(st_ref, gg_ref, be_ref)
    v = jnp.maximum(z_ref[...] * a + b, 0.0) * dv_ref[...]
    half = _H // 2
    oa_ref[...] = v[:, :half]
    ob_ref[...] = v[:, half:]


@functools.cache
def _bnrelu_scale_kernel(R):
    half = _H // 2
    return pl.pallas_call(
        _bnrelu_scale_body,
        grid=(_NP // R,),
        in_specs=[pl.BlockSpec((R, _H), lambda i: (i, 0)),
                  pl.BlockSpec((8, _H), lambda i: (0, 0)),
                  pl.BlockSpec((1, _H), lambda i: (0, 0)),
                  pl.BlockSpec((1, _H), lambda i: (0, 0)),
                  pl.BlockSpec((R, 1), lambda i: (i, 0))],
        out_specs=[pl.BlockSpec((R, half), lambda i: (i, 0)),
                   pl.BlockSpec((R, half), lambda i: (i, 0))],
        out_shape=[jax.ShapeDtypeStruct((_NP, half), jnp.float32),
                   jax.ShapeDtypeStruct((_NP, half), jnp.float32)],
    )


def _bnrelu_mm_body(z_ref, st_ref, gg_ref, be_ref, dv_ref, w_ref, o_ref):
    a, b = _bn_coeffs(st_ref, gg_ref, be_ref)
    xn = jnp.maximum(z_ref[...] * a + b, 0.0)
    o_ref[...] = jnp.dot(xn, w_ref[...],
                         preferred_element_type=jnp.float32) * dv_ref[...]


@functools.cache
def _bnrelu_mm_kernel(Din, Dout, R):
    return pl.pallas_call(
        _bnrelu_mm_body,
        grid=(_NP // R,),
        in_specs=[pl.BlockSpec((R, Din), lambda i: (i, 0)),
                  pl.BlockSpec((8, Din), lambda i: (0, 0)),
                  pl.BlockSpec((1, Din), lambda i: (0, 0)),
                  pl.BlockSpec((1, Din), lambda i: (0, 0)),
                  pl.BlockSpec((R, 1), lambda i: (i, 0)),
                  pl.BlockSpec((Din, Dout), lambda i: (0, 0))],
        out_specs=pl.BlockSpec((R, Dout), lambda i: (i, 0)),
        out_shape=jax.ShapeDtypeStruct((_NP, Dout), jnp.float32),
    )


def _stats_scaled_body(y_ref, dv_ref, st_ref):
    i = pl.program_id(0)
    z = y_ref[...] * dv_ref[...]

    @pl.when(i == 0)
    def _():
        st_ref[...] = jnp.zeros_like(st_ref)

    st_ref[0:1, :] = st_ref[0:1, :] + jnp.sum(z, axis=0, keepdims=True)
    st_ref[1:2, :] = st_ref[1:2, :] + jnp.sum(z * z, axis=0, keepdims=True)


@functools.cache
def _stats_scaled_kernel(D, R):
    return pl.pallas_call(
        _stats_scaled_body,
        grid=(_NP // R,),
        in_specs=[pl.BlockSpec((R, D), lambda i: (i, 0)),
                  pl.BlockSpec((R, 1), lambda i: (i, 0))],
        out_specs=pl.BlockSpec((8, D), lambda i: (0, 0)),
        out_shape=jax.ShapeDtypeStruct((8, D), jnp.float32),
    )


def _bnrelu_scaled_body(y_ref, st_ref, gg_ref, be_ref, dv_ref, o_ref):
    a, b = _bn_coeffs(st_ref, gg_ref, be_ref)
    o_ref[...] = jnp.maximum((y_ref[...] * dv_ref[...]) * a + b, 0.0)


@functools.cache
def _bnrelu_scaled_kernel(D, R):
    return pl.pallas_call(
        _bnrelu_scaled_body,
        grid=(_NP // R,),
        in_specs=[pl.BlockSpec((R, D), lambda i: (i, 0)),
                  pl.BlockSpec((8, D), lambda i: (0, 0)),
                  pl.BlockSpec((1, D), lambda i: (0, 0)),
                  pl.BlockSpec((1, D), lambda i: (0, 0)),
                  pl.BlockSpec((R, 1), lambda i: (i, 0))],
        out_specs=pl.BlockSpec((R, D), lambda i: (i, 0)),
        out_shape=jax.ShapeDtypeStruct((_NP, D), jnp.float32),
    )


def _final_body(pp_ref, cc_ref, o_ref):
    p = pp_ref[0, :_B, :] + pp_ref[1, :_B, :]
    cnt = cc_ref[0, :_B, :] + cc_ref[1, :_B, :]
    p = p / jnp.maximum(cnt, 1.0)
    nrm = jnp.sqrt(jnp.sum(p * p, axis=1, keepdims=True))
    o_ref[...] = p / jnp.maximum(nrm, 1e-12)


@functools.cache
def _final_kernel():
    return pl.pallas_call(
        _final_body,
        out_shape=jax.ShapeDtypeStruct((_B, _OUT), jnp.float32),
    )


def kernel(x, edge_index, batch, W1, b1, g1, be1, W2, b2, g2, be2, W3, b3, g3, be3):
    f32 = jnp.float32
    src = edge_index[0]
    dst = edge_index[1]

    xp = jnp.pad(x, ((0, _NP - _N), (0, 0)))
    padfill = (jnp.arange(_EPAD - _E) % 1024).astype(jnp.int32)
    src_pad = jnp.concatenate([src, padfill])
    dst_pad = jnp.concatenate([dst, _N + padfill])
    batch_pad = jnp.concatenate(
        [batch, (_B + (jnp.arange(_NP - _N) % 16)).astype(jnp.int32)])

    degp = _deg_kernel()(dst_pad.reshape(_EPAD // 128, 128))
    dinv = _dinv_kernel()(degp.reshape(_NC, _NP // 128, 128)).reshape(_NP, 1)

    msg = _msg_kernel(128, 12800)

    V1 = _rowscale_kernel(6400)(xp, dinv)
    G1 = msg(src_pad, dst_pad, V1)
    W1p = jnp.pad(W1, ((0, _OUT - _IN), (0, 0)))
    Z1, st1 = _mm_stats_kernel(_OUT, _H, 3200)(G1, dinv, W1p)

    V2a, V2b = _bnrelu_scale_kernel(3200)(Z1, st1, g1.reshape(1, _H),
                                          be1.reshape(1, _H), dinv)
    G2a = msg(src_pad, dst_pad, V2a)
    G2b = msg(src_pad, dst_pad, V2b)
    Z2, st2 = _mm2_stats_kernel(3200)(G2a, G2b, dinv, W2[:_H // 2],
                                      W2[_H // 2:])

    U3 = _bnrelu_mm_kernel(_H, _OUT, 3200)(Z2, st2, g2.reshape(1, _H),
                                           be2.reshape(1, _H), dinv, W3)
    Y3 = msg(src_pad, dst_pad, U3)
    st3 = _stats_scaled_kernel(_OUT, 6400)(Y3, dinv)
    X4 = _bnrelu_scaled_kernel(_OUT, 6400)(Y3, st3, g3.reshape(1, _OUT),
                                           be3.reshape(1, _OUT), dinv)

    poolp, cntp = _pool_kernel()(
        X4, batch_pad,
        jnp.zeros((_CNTP, _OUT), f32), jnp.zeros((_CNTC,), f32))
    return _final_kernel()(poolp.reshape(_NC, _CNTP, _OUT),
                           cntp.reshape(_NC, _CNTC, 1))
